# baseline (speedup 1.0000x reference)
import functools

import jax
import jax.numpy as jnp
from jax import lax
from jax.experimental import pallas as pl
from jax.experimental.pallas import tpu as pltpu
from jax.experimental.pallas import tpu_sc as plsc

_SMOOTH = 0.1
_IGNORE = 1
_N = 4096
_V = 32000
_R = 256
_VB = 16000

_NC = 2
_NS = 16
_NW = _NC * _NS
_L = 16
_CW = 128
_BW = _N // _NW


def _sc_gather_body(pred_hbm, tgt_hbm, out_hbm, tidx_v, sem):
    wid = lax.axis_index("s") * _NC + lax.axis_index("c")
    base = wid * _BW
    pltpu.sync_copy(tgt_hbm.at[pl.ds(base, _BW)], tidx_v)
    copies = []
    for c in range(_BW // _L):
        t16 = tidx_v[pl.ds(c * _L, _L)]
        for l in range(_L):
            b = c * _L + l
            o = pl.multiple_of(
                lax.shift_left(lax.shift_right_logical(t16[l], 7), 7), _CW)
            rr = pl.multiple_of(base + (b // 8) * 8, 8)
            copies.append(pltpu.async_copy(
                pred_hbm.at[pl.ds(rr, 8), pl.ds(o, _CW)],
                out_hbm.at[base + b], sem))
    for cp in copies:
        cp.wait()


@functools.partial(
    pl.kernel,
    mesh=plsc.VectorSubcoreMesh(core_axis_name="c", subcore_axis_name="s"),
    out_type=jax.ShapeDtypeStruct((_N, 8, _CW), jnp.float32),
    scratch_types=[
        pltpu.VMEM((_BW,), jnp.int32),
        pltpu.SemaphoreType.DMA,
    ],
)
def _sc_gather(pred_hbm, tgt_hbm, out_hbm, tidx_v, sem):
    _sc_gather_body(pred_hbm, tgt_hbm, out_hbm, tidx_v, sem)


def _stream_body(pred_ref, lse_ref, s_ref, m_sc, se_sc, s_sc):
    j = pl.program_id(1)
    nj = pl.num_programs(1)

    @pl.when(j == 0)
    def _init_row():
        m_sc[...] = jnp.full_like(m_sc, -jnp.inf)
        se_sc[...] = jnp.zeros_like(se_sc)
        s_sc[...] = jnp.zeros_like(s_sc)

    x = pred_ref[...]
    bm = jnp.max(x, axis=1, keepdims=True)
    m_old = m_sc[...]
    m_new = jnp.maximum(m_old, bm)
    alpha = jnp.exp(m_old - m_new)
    e = jnp.exp(x - m_new)
    se_sc[...] = se_sc[...] * alpha + jnp.sum(e, axis=1, keepdims=True)
    s_sc[...] += jnp.sum(x, axis=1, keepdims=True)
    m_sc[...] = m_new

    @pl.when(j == nj - 1)
    def _finish_row():
        lse_ref[...] = m_sc[...] + jnp.log(se_sc[...])
        s_ref[...] = s_sc[...]


def _combine_body(lse_ref, s_ref, ch_ref, tgt_ref, out_ref):
    lse = lse_ref[...]
    s = s_ref[...]
    ch = ch_ref[...]
    t = tgt_ref[...]
    t3 = t.reshape(_N, 1, 1)
    row = lax.broadcasted_iota(jnp.int32, ch.shape, 0)
    sub = lax.broadcasted_iota(jnp.int32, ch.shape, 1)
    lane = lax.broadcasted_iota(jnp.int32, ch.shape, 2)
    m = jnp.logical_and(sub == (row & 7), lane == (t3 & (_CW - 1)))
    tv = jnp.sum(jnp.sum(jnp.where(m, ch, 0.0), axis=2), axis=1,
                 keepdims=True)
    eps = _SMOOTH / (_V - 1)
    coef = 1.0 - _SMOOTH - eps
    loss = jnp.where(t != _IGNORE,
                     -(eps * (s - _V * lse) + coef * (tv - lse)), 0.0)
    out_ref[...] = (jnp.sum(loss) * (1.0 / _N)).reshape(1, 1)


def kernel(pred, target):
    tgt = target.astype(jnp.int32)
    ch = _sc_gather(pred, tgt)
    lse, s = pl.pallas_call(
        _stream_body,
        grid=(_N // _R, _V // _VB),
        in_specs=[pl.BlockSpec((_R, _VB), lambda i, j: (i, j))],
        out_specs=[
            pl.BlockSpec((_R, 1), lambda i, j: (i, 0)),
            pl.BlockSpec((_R, 1), lambda i, j: (i, 0)),
        ],
        out_shape=[
            jax.ShapeDtypeStruct((_N, 1), jnp.float32),
            jax.ShapeDtypeStruct((_N, 1), jnp.float32),
        ],
        scratch_shapes=[
            pltpu.VMEM((_R, 1), jnp.float32),
            pltpu.VMEM((_R, 1), jnp.float32),
            pltpu.VMEM((_R, 1), jnp.float32),
        ],
    )(pred)
    out = pl.pallas_call(
        _combine_body,
        out_shape=jax.ShapeDtypeStruct((1, 1), jnp.float32),
    )(lse, s, ch, tgt.reshape(_N, 1))
    return out[0, 0]


# speedup vs baseline: 2.8221x; 2.8221x over previous
import jax
import jax.numpy as jnp
from jax import lax
from jax.experimental import pallas as pl
from jax.experimental.pallas import tpu as pltpu

_SMOOTH = 0.1
_IGNORE = 1
_N = 4096
_V = 32000
_R = 256
_VB = 16000


def _loss_body(tgt_ref, pred_ref, out_ref, m_ref, se_ref, s_ref, tv_ref):
    i = pl.program_id(0)
    j = pl.program_id(1)
    nj = pl.num_programs(1)

    @pl.when(j == 0)
    def _init_row():
        m_ref[...] = jnp.full_like(m_ref, -jnp.inf)
        se_ref[...] = jnp.zeros_like(se_ref)
        s_ref[...] = jnp.zeros_like(s_ref)
        tv_ref[...] = jnp.zeros_like(tv_ref)

    @pl.when(jnp.logical_and(i == 0, j == 0))
    def _init_out():
        out_ref[...] = jnp.zeros_like(out_ref)

    x = pred_ref[...]
    t = tgt_ref[0, :, :]
    bm = jnp.max(x, axis=1, keepdims=True)
    m_old = m_ref[...]
    m_new = jnp.maximum(m_old, bm)
    alpha = jnp.exp(m_old - m_new)
    e = jnp.exp(x - m_new)
    se_ref[...] = se_ref[...] * alpha + jnp.sum(e, axis=1, keepdims=True)
    s_ref[...] += jnp.sum(x, axis=1, keepdims=True)
    col = lax.broadcasted_iota(jnp.int32, x.shape, 1) + j * _VB
    tv_ref[...] += jnp.sum(jnp.where(col == t, x, 0.0), axis=1, keepdims=True)
    m_ref[...] = m_new

    @pl.when(j == nj - 1)
    def _finish_row():
        lse = m_ref[...] + jnp.log(se_ref[...])
        sum_logp = s_ref[...] - _V * lse
        logp_t = tv_ref[...] - lse
        eps = _SMOOTH / (_V - 1)
        coef = 1.0 - _SMOOTH - eps
        loss = jnp.where(t != _IGNORE, -(eps * sum_logp + coef * logp_t), 0.0)
        out_ref[...] += jnp.sum(loss).reshape(1, 1)


def kernel(pred, target):
    tgt3 = target.astype(jnp.int32).reshape(_N // _R, _R, 1)
    out = pl.pallas_call(
        _loss_body,
        grid=(_N // _R, _V // _VB),
        in_specs=[
            pl.BlockSpec((1, _R, 1), lambda i, j: (i, 0, 0)),
            pl.BlockSpec((_R, _VB), lambda i, j: (i, j)),
        ],
        out_specs=pl.BlockSpec((1, 1), lambda i, j: (0, 0)),
        out_shape=jax.ShapeDtypeStruct((1, 1), jnp.float32),
        scratch_shapes=[
            pltpu.VMEM((_R, 1), jnp.float32),
            pltpu.VMEM((_R, 1), jnp.float32),
            pltpu.VMEM((_R, 1), jnp.float32),
            pltpu.VMEM((_R, 1), jnp.float32),
        ],
    )(tgt3, pred)
    return out[0, 0] / _N
